# Initial kernel scaffold; baseline (speedup 1.0000x reference)
import jax
import jax.numpy as jnp
from jax.experimental import pallas as pl


def kernel(x, edge_index, edge_attr, batch, Wi_n, Wi_e, Wg_n, Wg_e, bg, W1, b1, W2, b2):
    raise NotImplementedError("write your pallas kernel here")


# speedup vs baseline: 3.3979x; 3.3979x over previous
import functools

import jax
import jax.numpy as jnp
from jax import lax
from jax.experimental import pallas as pl
from jax.experimental.pallas import tpu as pltpu
from jax.experimental.pallas import tpu_sc as plsc

N = 10000
E = 160000
D = 300
DP = 320
DH = DP // 2
DE = 16
G = 512

NSUB = 16
NP = 10240
ROWS = NP // NSUB
CH = 40
EPW = E // NSUB
NCH = EPW // CH
U = 2
GROUPS = NCH // U

NB = 1000
NGRID = N // NB


def _sc_body(xl, xr, src2d, dst2d, ea, zx, z16, ones_hbm,
             x_out, a_out, d_out,
             acc_x, acc_ad, src_g, dst_g, rows_v, ea_v, one_v, gsem):
    c = lax.axis_index("c")
    s = lax.axis_index("s")
    r0 = s * ROWS

    pltpu.sync_copy(zx.at[pl.ds(r0, ROWS)], acc_x.at[pl.ds(r0, ROWS)])
    pltpu.sync_copy(z16.at[pl.ds(r0, ROWS)], acc_ad.at[pl.ds(r0, ROWS)])
    pltpu.sync_copy(ones_hbm, one_v)
    plsc.subcore_barrier()

    def run_phase(xtab, is_core0):
        def group(g, _):
            pltpu.sync_copy(src2d.at[s, g], src_g)
            pltpu.sync_copy(dst2d.at[s, g], dst_g)
            for ji in range(U):
                pltpu.async_copy(xtab.at[src_g.at[ji]],
                                 rows_v.at[ji], gsem)
            if is_core0:
                pltpu.sync_copy(
                    ea.at[pl.ds(s * EPW + g * (U * CH), U * CH)], ea_v)
            for ji in range(U):
                pltpu.make_async_copy(xtab.at[src_g.at[ji]],
                                      rows_v.at[ji], gsem).wait()
            for ji in range(U):
                pltpu.sync_copy(rows_v.at[ji],
                                acc_x.at[dst_g.at[ji]], add=True)
                if is_core0:
                    pltpu.sync_copy(ea_v.at[pl.ds(ji * CH, CH)],
                                    acc_ad.at[dst_g.at[ji]], add=True)
                else:
                    pltpu.sync_copy(one_v,
                                    acc_ad.at[dst_g.at[ji]], add=True)
            return 0

        lax.fori_loop(0, GROUPS, group, 0)

    @pl.when(c == 0)
    def _():
        run_phase(xl, True)

    @pl.when(c == 1)
    def _():
        run_phase(xr, False)

    plsc.subcore_barrier()

    pltpu.sync_copy(acc_x.at[pl.ds(r0, ROWS)],
                    x_out.at[c, pl.ds(r0, ROWS)])

    @pl.when(c == 0)
    def _():
        pltpu.sync_copy(acc_ad.at[pl.ds(r0, ROWS)], a_out.at[pl.ds(r0, ROWS)])

    @pl.when(c == 1)
    def _():
        pltpu.sync_copy(acc_ad.at[pl.ds(r0, ROWS)], d_out.at[pl.ds(r0, ROWS)])


def _sc_aggregate(xl, xr, src2d, dst2d, ea, zx, z16, ones_hbm):
    mesh = plsc.VectorSubcoreMesh(core_axis_name="c", subcore_axis_name="s")
    f = pl.kernel(
        _sc_body,
        out_type=(
            jax.ShapeDtypeStruct((2, NP, DH), jnp.float32),
            jax.ShapeDtypeStruct((NP, DE), jnp.float32),
            jax.ShapeDtypeStruct((NP, DE), jnp.float32),
        ),
        mesh=mesh,
        compiler_params=pltpu.CompilerParams(use_tc_tiling_on_sc=False),
        scratch_types=(
            pltpu.VMEM_SHARED((NP, DH), jnp.float32),
            pltpu.VMEM_SHARED((NP, DE), jnp.float32),
            pltpu.VMEM((U, CH), jnp.int32),
            pltpu.VMEM((U, CH), jnp.int32),
            pltpu.VMEM((U, CH, DH), jnp.float32),
            pltpu.VMEM((U * CH, DE), jnp.float32),
            pltpu.VMEM((CH, DE), jnp.float32),
            pltpu.SemaphoreType.DMA,
        ),
    )
    return f(xl, xr, src2d, dst2d, ea, zx, z16, ones_hbm)


def _tc_body(x0_ref, x1_ref, a_ref, d_ref, bt_ref,
             wgp_ref, wge_ref, bg_ref, wip_ref, wie_ref,
             w1_ref, b1_ref, w2_ref, b2_ref,
             z_ref, segmax_s, counts_s, p0_s, p1_s):
    i = pl.program_id(0)

    @pl.when(i == 0)
    def _():
        segmax_s[...] = jnp.full((1, G), -jnp.inf, jnp.float32)
        counts_s[...] = jnp.zeros((1, G), jnp.float32)
        p0_s[...] = jnp.zeros((G, D), jnp.float32)
        p1_s[...] = jnp.zeros((G, D), jnp.float32)

    x0 = x0_ref[0]
    x1 = x1_ref[0]
    ae = a_ref[...]
    dg = d_ref[...][:, 0:1]
    btc = bt_ref[0]

    wgp = wgp_ref[...]
    h = (jnp.dot(x0, wgp[:DH], preferred_element_type=jnp.float32)
         + jnp.dot(x1, wgp[DH:], preferred_element_type=jnp.float32)
         + jnp.dot(ae, wge_ref[...], preferred_element_type=jnp.float32)
         + dg * bg_ref[...])
    h = jnp.maximum(h, 0.0)

    wip = wip_ref[...]
    imp_pre = (jnp.dot(x0, wip[:DH], preferred_element_type=jnp.float32)
               + jnp.dot(x1, wip[DH:], preferred_element_type=jnp.float32)
               + jnp.dot(ae, wie_ref[...], preferred_element_type=jnp.float32))
    imp = jnp.maximum(imp_pre, 0.0) + jnp.log1p(jnp.exp(-jnp.abs(imp_pre)))

    seg = lax.broadcasted_iota(jnp.int32, (1, G), 1)
    mask = btc == seg
    maskf = mask.astype(jnp.float32)

    counts_s[...] += jnp.sum(maskf, axis=0, keepdims=True)
    m = jnp.max(jnp.where(mask, imp, -jnp.inf), axis=0, keepdims=True)
    segmax_s[...] = jnp.maximum(segmax_s[...], m)

    dn = (((0,), (0,)), ((), ()))
    p0_s[...] += lax.dot_general(maskf, h, dn,
                                 preferred_element_type=jnp.float32)
    p1_s[...] += lax.dot_general(maskf, h * imp, dn,
                                 preferred_element_type=jnp.float32)

    @pl.when(i == NGRID - 1)
    def _():
        om = segmax_s[...]
        cnt = jnp.maximum(counts_s[...], 1.0)
        stacked = jnp.concatenate([1.0 / (10.0 * om), 1.0 / cnt], axis=0)
        eye = (lax.broadcasted_iota(jnp.int32, (G, G), 0)
               == lax.broadcasted_iota(jnp.int32, (G, G), 1)
               ).astype(jnp.float32)
        cols = lax.dot_general(eye, stacked, (((1,), (1,)), ((), ())),
                               precision=lax.Precision.HIGHEST,
                               preferred_element_type=jnp.float32)
        inv10om = cols[:, 0:1]
        invcnt = cols[:, 1:2]
        pooled = (p1_s[...] * inv10om + 0.9 * p0_s[...]) * invcnt
        z1 = jnp.maximum(
            jnp.dot(pooled, w1_ref[...], preferred_element_type=jnp.float32)
            + b1_ref[...], 0.0)
        z_ref[...] = (jnp.dot(z1, w2_ref[...],
                              preferred_element_type=jnp.float32)
                      + b2_ref[...])


def _tc_dense(x_parts, a_e, deg16, batch3, wgp, wge, bg2, wip, wie,
              w1, b1r, w2, b2r):
    full = lambda shp: pl.BlockSpec(shp, lambda i: (0,) * len(shp))
    return pl.pallas_call(
        _tc_body,
        grid=(NGRID,),
        in_specs=[
            pl.BlockSpec((1, NB, DH), lambda i: (0, i, 0)),
            pl.BlockSpec((1, NB, DH), lambda i: (1, i, 0)),
            pl.BlockSpec((NB, DE), lambda i: (i, 0)),
            pl.BlockSpec((NB, DE), lambda i: (i, 0)),
            pl.BlockSpec((1, NB, 1), lambda i: (i, 0, 0)),
            full((DP, D)), full((DE, D)), full((1, D)),
            full((DP, 1)), full((DE, 1)),
            full((D, D)), full((1, D)), full((D, D)), full((1, D)),
        ],
        out_specs=pl.BlockSpec((G, D), lambda i: (0, 0)),
        out_shape=jax.ShapeDtypeStruct((G, D), jnp.float32),
        scratch_shapes=[
            pltpu.VMEM((1, G), jnp.float32),
            pltpu.VMEM((1, G), jnp.float32),
            pltpu.VMEM((G, D), jnp.float32),
            pltpu.VMEM((G, D), jnp.float32),
        ],
    )(x_parts, x_parts, a_e, deg16, batch3, wgp, wge, bg2, wip, wie,
      w1, b1r, w2, b2r)


def kernel(x, edge_index, edge_attr, batch, Wi_n, Wi_e, Wg_n, Wg_e, bg,
           W1, b1, W2, b2):
    xl = x[:, :DH]
    xr = jnp.pad(x[:, DH:], ((0, 0), (0, DP - D)))
    src2d = edge_index[0].reshape(NSUB, GROUPS, U, CH)
    dst2d = edge_index[1].reshape(NSUB, GROUPS, U, CH)
    zx = jnp.zeros((NP, DH), jnp.float32)
    z16 = jnp.zeros((NP, DE), jnp.float32)
    ones_hbm = jnp.ones((CH, DE), jnp.float32)

    x_parts, a_e, deg16 = _sc_aggregate(
        xl, xr, src2d, dst2d, edge_attr, zx, z16, ones_hbm)

    wgp = jnp.pad(Wg_n, ((0, DP - D), (0, 0)))
    wip = jnp.pad(Wi_n, ((0, DP - D), (0, 0)))
    batch3 = batch.reshape(NGRID, NB, 1)

    return _tc_dense(x_parts, a_e, deg16, batch3, wgp, Wg_e,
                     bg.reshape(1, D), wip, Wi_e,
                     W1, b1.reshape(1, D), W2, b2.reshape(1, D))


# speedup vs baseline: 5.7736x; 1.6992x over previous
import functools

import jax
import jax.numpy as jnp
from jax import lax
from jax.experimental import pallas as pl
from jax.experimental.pallas import tpu as pltpu
from jax.experimental.pallas import tpu_sc as plsc

N = 10000
E = 160000
D = 300
DP = 320
DH = DP // 2
DE = 16
G = 512

NSUB = 16
NP = 10240
ROWS = NP // NSUB
CH = 80
EPW = E // NSUB
NCH = EPW // CH
NBUF = 5
OUTER = NCH // NBUF
EAG = NBUF * CH

NB = 1000
NGRID = N // NB


def _sc_body(xl, xr, idx3, ea, zx, z16, ones_hbm,
             x_out, a_out, d_out,
             acc_x, acc_ad, idx_v, rows_v, ea_v, one_v,
             sem0, sem1, sem2, sem3, sem4):
    c = lax.axis_index("c")
    s = lax.axis_index("s")
    r0 = s * ROWS
    gsems = (sem0, sem1, sem2, sem3, sem4)

    pltpu.sync_copy(zx.at[pl.ds(r0, ROWS)], acc_x.at[pl.ds(r0, ROWS)])
    pltpu.sync_copy(z16.at[pl.ds(r0, ROWS)], acc_ad.at[pl.ds(r0, ROWS)])
    pltpu.sync_copy(idx3.at[s], idx_v)
    pltpu.sync_copy(ones_hbm, one_v)
    plsc.subcore_barrier()

    def run_phase(xtab, is_core0):
        for b in range(NBUF):
            pltpu.async_copy(xtab.at[idx_v.at[b, 0]], rows_v.at[b], gsems[b])
        if is_core0:
            pltpu.sync_copy(ea.at[pl.ds(s * EPW, EAG)], ea_v)

        def outer(p, _):
            g0 = p * NBUF
            for ji in range(NBUF):
                g = g0 + ji
                pltpu.make_async_copy(xtab.at[idx_v.at[g, 0]],
                                      rows_v.at[ji], gsems[ji]).wait()
                pltpu.sync_copy(rows_v.at[ji],
                                acc_x.at[idx_v.at[g, 1]], add=True)
                if is_core0:
                    pltpu.sync_copy(ea_v.at[pl.ds(ji * CH, CH)],
                                    acc_ad.at[idx_v.at[g, 1]], add=True)
                else:
                    pltpu.sync_copy(one_v,
                                    acc_ad.at[idx_v.at[g, 1]], add=True)
                gn = g + NBUF

                @pl.when(gn < NCH)
                def _():
                    pltpu.async_copy(xtab.at[idx_v.at[gn, 0]],
                                     rows_v.at[ji], gsems[ji])
            if is_core0:
                @pl.when(p + 1 < OUTER)
                def _():
                    pltpu.sync_copy(
                        ea.at[pl.ds(s * EPW + (p + 1) * EAG, EAG)], ea_v)
            return 0

        lax.fori_loop(0, OUTER, outer, 0)

    @pl.when(c == 0)
    def _():
        run_phase(xl, True)

    @pl.when(c == 1)
    def _():
        run_phase(xr, False)

    plsc.subcore_barrier()

    pltpu.sync_copy(acc_x.at[pl.ds(r0, ROWS)],
                    x_out.at[c, pl.ds(r0, ROWS)])

    @pl.when(c == 0)
    def _():
        pltpu.sync_copy(acc_ad.at[pl.ds(r0, ROWS)], a_out.at[pl.ds(r0, ROWS)])

    @pl.when(c == 1)
    def _():
        pltpu.sync_copy(acc_ad.at[pl.ds(r0, ROWS)], d_out.at[pl.ds(r0, ROWS)])


def _sc_aggregate(xl, xr, idx3, ea, zx, z16, ones_hbm):
    mesh = plsc.VectorSubcoreMesh(core_axis_name="c", subcore_axis_name="s")
    f = pl.kernel(
        _sc_body,
        out_type=(
            jax.ShapeDtypeStruct((2, NP, DH), jnp.bfloat16),
            jax.ShapeDtypeStruct((NP, DE), jnp.float32),
            jax.ShapeDtypeStruct((NP, DE), jnp.float32),
        ),
        mesh=mesh,
        compiler_params=pltpu.CompilerParams(use_tc_tiling_on_sc=False),
        scratch_types=(
            pltpu.VMEM_SHARED((NP, DH), jnp.bfloat16),
            pltpu.VMEM_SHARED((NP, DE), jnp.float32),
            pltpu.VMEM((NCH, 2, CH), jnp.int32),
            pltpu.VMEM((NBUF, CH, DH), jnp.bfloat16),
            pltpu.VMEM((EAG, DE), jnp.float32),
            pltpu.VMEM((CH, DE), jnp.float32),
            pltpu.SemaphoreType.DMA,
            pltpu.SemaphoreType.DMA,
            pltpu.SemaphoreType.DMA,
            pltpu.SemaphoreType.DMA,
            pltpu.SemaphoreType.DMA,
        ),
    )
    return f(xl, xr, idx3, ea, zx, z16, ones_hbm)


def _tc_body(x0_ref, x1_ref, a_ref, d_ref, bt_ref,
             wgp_ref, wge_ref, bg_ref, wip_ref, wie_ref,
             w1_ref, b1_ref, w2_ref, b2_ref,
             z_ref, segmax_s, counts_s, p0_s, p1_s):
    i = pl.program_id(0)

    @pl.when(i == 0)
    def _():
        segmax_s[...] = jnp.full((1, G), -jnp.inf, jnp.float32)
        counts_s[...] = jnp.zeros((1, G), jnp.float32)
        p0_s[...] = jnp.zeros((G, D), jnp.float32)
        p1_s[...] = jnp.zeros((G, D), jnp.float32)

    x0 = x0_ref[0]
    x1 = x1_ref[0]
    ae = a_ref[...]
    dg = d_ref[...][:, 0:1]
    btc = bt_ref[0]

    wgp = wgp_ref[...]
    h = (jnp.dot(x0, wgp[:DH], preferred_element_type=jnp.float32)
         + jnp.dot(x1, wgp[DH:], preferred_element_type=jnp.float32)
         + jnp.dot(ae, wge_ref[...], preferred_element_type=jnp.float32)
         + dg * bg_ref[...])
    h = jnp.maximum(h, 0.0)

    wip = wip_ref[...]
    imp_pre = (jnp.dot(x0, wip[:DH], preferred_element_type=jnp.float32)
               + jnp.dot(x1, wip[DH:], preferred_element_type=jnp.float32)
               + jnp.dot(ae, wie_ref[...], preferred_element_type=jnp.float32))
    imp = jnp.maximum(imp_pre, 0.0) + jnp.log1p(jnp.exp(-jnp.abs(imp_pre)))

    seg = lax.broadcasted_iota(jnp.int32, (1, G), 1)
    mask = btc == seg
    maskf = mask.astype(jnp.float32)

    counts_s[...] += jnp.sum(maskf, axis=0, keepdims=True)
    m = jnp.max(jnp.where(mask, imp, -jnp.inf), axis=0, keepdims=True)
    segmax_s[...] = jnp.maximum(segmax_s[...], m)

    dn = (((0,), (0,)), ((), ()))
    p0_s[...] += lax.dot_general(maskf, h, dn,
                                 preferred_element_type=jnp.float32)
    p1_s[...] += lax.dot_general(maskf, h * imp, dn,
                                 preferred_element_type=jnp.float32)

    @pl.when(i == NGRID - 1)
    def _():
        om = segmax_s[...]
        cnt = jnp.maximum(counts_s[...], 1.0)
        stacked = jnp.concatenate([1.0 / (10.0 * om), 1.0 / cnt], axis=0)
        eye = (lax.broadcasted_iota(jnp.int32, (G, G), 0)
               == lax.broadcasted_iota(jnp.int32, (G, G), 1)
               ).astype(jnp.float32)
        cols = lax.dot_general(eye, stacked, (((1,), (1,)), ((), ())),
                               precision=lax.Precision.HIGHEST,
                               preferred_element_type=jnp.float32)
        inv10om = cols[:, 0:1]
        invcnt = cols[:, 1:2]
        pooled = (p1_s[...] * inv10om + 0.9 * p0_s[...]) * invcnt
        z1 = jnp.maximum(
            jnp.dot(pooled, w1_ref[...], preferred_element_type=jnp.float32)
            + b1_ref[...], 0.0)
        z_ref[...] = (jnp.dot(z1, w2_ref[...],
                              preferred_element_type=jnp.float32)
                      + b2_ref[...])


def _tc_dense(x_parts, a_e, deg16, batch3, wgp, wge, bg2, wip, wie,
              w1, b1r, w2, b2r):
    full = lambda shp: pl.BlockSpec(shp, lambda i: (0,) * len(shp))
    return pl.pallas_call(
        _tc_body,
        grid=(NGRID,),
        in_specs=[
            pl.BlockSpec((1, NB, DH), lambda i: (0, i, 0)),
            pl.BlockSpec((1, NB, DH), lambda i: (1, i, 0)),
            pl.BlockSpec((NB, DE), lambda i: (i, 0)),
            pl.BlockSpec((NB, DE), lambda i: (i, 0)),
            pl.BlockSpec((1, NB, 1), lambda i: (i, 0, 0)),
            full((DP, D)), full((DE, D)), full((1, D)),
            full((DP, 1)), full((DE, 1)),
            full((D, D)), full((1, D)), full((D, D)), full((1, D)),
        ],
        out_specs=pl.BlockSpec((G, D), lambda i: (0, 0)),
        out_shape=jax.ShapeDtypeStruct((G, D), jnp.float32),
        scratch_shapes=[
            pltpu.VMEM((1, G), jnp.float32),
            pltpu.VMEM((1, G), jnp.float32),
            pltpu.VMEM((G, D), jnp.float32),
            pltpu.VMEM((G, D), jnp.float32),
        ],
    )(x_parts, x_parts, a_e, deg16, batch3, wgp, wge, bg2, wip, wie,
      w1, b1r, w2, b2r)


def kernel(x, edge_index, edge_attr, batch, Wi_n, Wi_e, Wg_n, Wg_e, bg,
           W1, b1, W2, b2):
    xb = x.astype(jnp.bfloat16)
    xl = xb[:, :DH]
    xr = jnp.pad(xb[:, DH:], ((0, 0), (0, DP - D)))
    idx3 = jnp.stack([edge_index[0].reshape(NSUB, NCH, CH),
                      edge_index[1].reshape(NSUB, NCH, CH)], axis=2)
    zx = jnp.zeros((NP, DH), jnp.bfloat16)
    z16 = jnp.zeros((NP, DE), jnp.float32)
    ones_hbm = jnp.ones((CH, DE), jnp.float32)

    x_parts, a_e, deg16 = _sc_aggregate(
        xl, xr, idx3, edge_attr, zx, z16, ones_hbm)

    wgp = jnp.pad(Wg_n, ((0, DP - D), (0, 0)))
    wip = jnp.pad(Wi_n, ((0, DP - D), (0, 0)))
    batch3 = batch.reshape(NGRID, NB, 1)

    return _tc_dense(x_parts, a_e, deg16, batch3, wgp, Wg_e,
                     bg.reshape(1, D), wip, Wi_e,
                     W1, b1.reshape(1, D), W2, b2.reshape(1, D))
